# initial kernel scaffold (unmeasured)
import jax
import jax.numpy as jnp
from jax import lax
from jax.experimental import pallas as pl
from jax.experimental.pallas import tpu as pltpu

N_Z = 4


def kernel(x, router, W1, W2):
    t_loc, d_model = x.shape
    e_loc, _, f_dim = W1.shape
    n_tok = N_Z * t_loc
    n_exp = N_Z * e_loc

    def body(x_ref, router_ref, w1_hbm, w2_hbm, out_ref,
             x_full, router_all, w1_buf, w2_buf, rs_send, rs_recv,
             ag_x_send, ag_x_recv, ag_r_send, ag_r_recv,
             rs_send_sems, rs_recv_sems, w1_sem, w2_sem):
        zi = lax.axis_index("z")
        xi = lax.axis_index("x")
        yi = lax.axis_index("y")
        left = (zi - 1) % N_Z
        right = (zi + 1) % N_Z

        barrier = pltpu.get_barrier_semaphore()
        for nbr in (left, right):
            pl.semaphore_signal(
                barrier, inc=1,
                device_id=(xi, yi, nbr),
                device_id_type=pl.DeviceIdType.MESH,
            )
        pl.semaphore_wait(barrier, 2)

        cp1 = pltpu.make_async_copy(w1_hbm.at[0], w1_buf, w1_sem)
        cp2 = pltpu.make_async_copy(w2_hbm.at[0], w2_buf, w2_sem)
        cp1.start()
        cp2.start()

        x_full[pl.ds(zi * t_loc, t_loc)] = x_ref[...]
        router_all[zi] = router_ref[...]

        for h in range(N_Z - 1):
            o = (zi - h) % N_Z
            rx = pltpu.make_async_remote_copy(
                src_ref=x_full.at[pl.ds(o * t_loc, t_loc)],
                dst_ref=x_full.at[pl.ds(o * t_loc, t_loc)],
                send_sem=ag_x_send.at[h],
                recv_sem=ag_x_recv.at[h],
                device_id=(xi, yi, right),
                device_id_type=pl.DeviceIdType.MESH,
            )
            rr = pltpu.make_async_remote_copy(
                src_ref=router_all.at[o],
                dst_ref=router_all.at[o],
                send_sem=ag_r_send.at[h],
                recv_sem=ag_r_recv.at[h],
                device_id=(xi, yi, right),
                device_id_type=pl.DeviceIdType.MESH,
            )
            rx.start()
            rr.start()
            rx.wait()
            rr.wait()

        xf = x_full[...]
        router_full = jnp.concatenate(
            [router_all[i] for i in range(N_Z)], axis=1)
        gates = jnp.dot(xf, router_full,
                        preferred_element_type=jnp.float32)
        iota = lax.broadcasted_iota(jnp.int32, (n_tok, n_exp), 1)
        m1 = jnp.max(gates, axis=1, keepdims=True)
        i1 = jnp.min(jnp.where(gates == m1, iota, n_exp),
                     axis=1, keepdims=True)
        neg = jnp.finfo(jnp.float32).min
        g2 = jnp.where(iota == i1, neg, gates)
        m2 = jnp.max(g2, axis=1, keepdims=True)
        i2 = jnp.min(jnp.where(g2 == m2, iota, n_exp),
                     axis=1, keepdims=True)
        e2 = jnp.exp(m2 - m1)
        w_top1 = 1.0 / (1.0 + e2)
        w_top2 = e2 / (1.0 + e2)

        acc = jnp.zeros((n_tok, d_model), jnp.float32)
        for j in range(e_loc):
            cp1.wait()
            cp2.wait()
            w1j = w1_buf[...]
            w2j = w2_buf[...]
            if j + 1 < e_loc:
                cp1 = pltpu.make_async_copy(w1_hbm.at[j + 1], w1_buf, w1_sem)
                cp2 = pltpu.make_async_copy(w2_hbm.at[j + 1], w2_buf, w2_sem)
            e_g = zi * e_loc + j
            wj = (jnp.where(i1 == e_g, w_top1, 0.0)
                  + jnp.where(i2 == e_g, w_top2, 0.0))
            h_act = jnp.maximum(
                jnp.dot(xf, w1j, preferred_element_type=jnp.float32), 0.0)
            acc = acc + jnp.dot(
                h_act, w2j, preferred_element_type=jnp.float32) * wj
            if j + 1 < e_loc:
                cp1.start()
                cp2.start()

        for s in range(N_Z - 1):
            c = (zi - 1 - s) % N_Z
            data = lax.dynamic_slice(acc, (c * t_loc, 0), (t_loc, d_model))
            if s > 0:
                data = data + rs_recv[s - 1]
            rs_send[...] = data
            r = pltpu.make_async_remote_copy(
                src_ref=rs_send,
                dst_ref=rs_recv.at[s],
                send_sem=rs_send_sems.at[s],
                recv_sem=rs_recv_sems.at[s],
                device_id=(xi, yi, right),
                device_id_type=pl.DeviceIdType.MESH,
            )
            r.start()
            r.wait()

        out_ref[...] = (
            lax.dynamic_slice(acc, (zi * t_loc, 0), (t_loc, d_model))
            + rs_recv[N_Z - 2])

    return pl.pallas_call(
        body,
        out_shape=jax.ShapeDtypeStruct((t_loc, d_model), jnp.float32),
        in_specs=[
            pl.BlockSpec(memory_space=pltpu.VMEM),
            pl.BlockSpec(memory_space=pltpu.VMEM),
            pl.BlockSpec(memory_space=pltpu.ANY),
            pl.BlockSpec(memory_space=pltpu.ANY),
        ],
        out_specs=pl.BlockSpec(memory_space=pltpu.VMEM),
        scratch_shapes=[
            pltpu.VMEM((n_tok, d_model), jnp.float32),
            pltpu.VMEM((N_Z, n_tok, e_loc), jnp.float32),
            pltpu.VMEM((d_model, f_dim), jnp.float32),
            pltpu.VMEM((f_dim, d_model), jnp.float32),
            pltpu.VMEM((t_loc, d_model), jnp.float32),
            pltpu.VMEM((N_Z - 1, t_loc, d_model), jnp.float32),
            pltpu.SemaphoreType.DMA((N_Z - 1,)),
            pltpu.SemaphoreType.DMA((N_Z - 1,)),
            pltpu.SemaphoreType.DMA((N_Z - 1,)),
            pltpu.SemaphoreType.DMA((N_Z - 1,)),
            pltpu.SemaphoreType.DMA((N_Z - 1,)),
            pltpu.SemaphoreType.DMA((N_Z - 1,)),
            pltpu.SemaphoreType.DMA,
            pltpu.SemaphoreType.DMA,
        ],
        compiler_params=pltpu.CompilerParams(collective_id=0),
    )(x, router, W1, W2)


# baseline (device time: 159846 ns/iter reference)
import jax
import jax.numpy as jnp
from jax import lax
from jax.experimental import pallas as pl
from jax.experimental.pallas import tpu as pltpu

N_Z = 4
F_CHUNK = 512


def kernel(x, router, W1, W2):
    t_loc, d_model = x.shape
    e_loc, _, f_dim = W1.shape
    n_tok = N_Z * t_loc
    n_exp = N_Z * e_loc

    def body(x_hbm, router_hbm, w1_hbm, w2_hbm, out_ref,
             x_full, router_all, w1_buf, w2_buf, acc_ref, rs_send, rs_recv,
             ag_x_send, ag_x_recv, ag_r_send, ag_r_recv,
             rs_send_sems, rs_recv_sems, in_x_sem, in_r_sem, w1_sem, w2_sem):
        zi = lax.axis_index("z")
        xi = lax.axis_index("x")
        yi = lax.axis_index("y")
        left = (zi - 1) % N_Z
        right = (zi + 1) % N_Z

        cpx = pltpu.make_async_copy(
            x_hbm, x_full.at[pl.ds(zi * t_loc, t_loc)], in_x_sem)
        cpr = pltpu.make_async_copy(router_hbm, router_all.at[zi], in_r_sem)
        cp1 = pltpu.make_async_copy(w1_hbm.at[0], w1_buf.at[0], w1_sem.at[0])
        cp2 = pltpu.make_async_copy(w2_hbm.at[0], w2_buf.at[0], w2_sem.at[0])
        cpx.start()
        cpr.start()
        cp1.start()
        cp2.start()

        barrier = pltpu.get_barrier_semaphore()
        for nbr in (left, right):
            pl.semaphore_signal(
                barrier, inc=1,
                device_id=(xi, yi, nbr),
                device_id_type=pl.DeviceIdType.MESH,
            )
        pl.semaphore_wait(barrier, 2)
        cpx.wait()
        cpr.wait()

        for h in range(N_Z - 1):
            o = (zi - h) % N_Z
            rx = pltpu.make_async_remote_copy(
                src_ref=x_full.at[pl.ds(o * t_loc, t_loc)],
                dst_ref=x_full.at[pl.ds(o * t_loc, t_loc)],
                send_sem=ag_x_send.at[h],
                recv_sem=ag_x_recv.at[h],
                device_id=(xi, yi, right),
                device_id_type=pl.DeviceIdType.MESH,
            )
            rr = pltpu.make_async_remote_copy(
                src_ref=router_all.at[o],
                dst_ref=router_all.at[o],
                send_sem=ag_r_send.at[h],
                recv_sem=ag_r_recv.at[h],
                device_id=(xi, yi, right),
                device_id_type=pl.DeviceIdType.MESH,
            )
            rx.start()
            rr.start()
            rx.wait()
            rr.wait()

        router_full = jnp.concatenate(
            [router_all[i] for i in range(N_Z)], axis=1)
        gates = jnp.dot(x_full[...], router_full,
                        preferred_element_type=jnp.float32,
                        precision=lax.Precision.HIGHEST)
        iota = lax.broadcasted_iota(jnp.int32, (n_tok, n_exp), 1)
        m1 = jnp.max(gates, axis=1, keepdims=True)
        i1 = jnp.min(jnp.where(gates == m1, iota, n_exp),
                     axis=1, keepdims=True)
        neg = jnp.finfo(jnp.float32).min
        g2 = jnp.where(iota == i1, neg, gates)
        m2 = jnp.max(g2, axis=1, keepdims=True)
        i2 = jnp.min(jnp.where(g2 == m2, iota, n_exp),
                     axis=1, keepdims=True)
        e2 = jnp.exp(m2 - m1)
        w_top1 = 1.0 / (1.0 + e2)
        w_top2 = e2 / (1.0 + e2)

        for j in range(e_loc):
            slot = j % 2
            cp1.wait()
            cp2.wait()
            if j + 1 < e_loc:
                nxt = (j + 1) % 2
                cp1 = pltpu.make_async_copy(
                    w1_hbm.at[j + 1], w1_buf.at[nxt], w1_sem.at[nxt])
                cp2 = pltpu.make_async_copy(
                    w2_hbm.at[j + 1], w2_buf.at[nxt], w2_sem.at[nxt])
                cp1.start()
                cp2.start()
            e_g = zi * e_loc + j
            wj = (jnp.where(i1 == e_g, w_top1, 0.0)
                  + jnp.where(i2 == e_g, w_top2, 0.0))
            for fc in range(0, f_dim, F_CHUNK):
                h_act = jnp.maximum(
                    jnp.dot(x_full[...], w1_buf[slot, :, fc:fc + F_CHUNK],
                            preferred_element_type=jnp.float32), 0.0)
                contrib = jnp.dot(
                    h_act, w2_buf[slot, fc:fc + F_CHUNK, :],
                    preferred_element_type=jnp.float32) * wj
                if j == 0 and fc == 0:
                    acc_ref[...] = contrib
                else:
                    acc_ref[...] = acc_ref[...] + contrib

        for s in range(N_Z - 1):
            c = (zi - 1 - s) % N_Z
            data = acc_ref[pl.ds(c * t_loc, t_loc)]
            if s > 0:
                data = data + rs_recv[s - 1]
            rs_send[...] = data
            r = pltpu.make_async_remote_copy(
                src_ref=rs_send,
                dst_ref=rs_recv.at[s],
                send_sem=rs_send_sems.at[s],
                recv_sem=rs_recv_sems.at[s],
                device_id=(xi, yi, right),
                device_id_type=pl.DeviceIdType.MESH,
            )
            r.start()
            r.wait()

        out_ref[...] = (
            acc_ref[pl.ds(zi * t_loc, t_loc)] + rs_recv[N_Z - 2])

    return pl.pallas_call(
        body,
        out_shape=jax.ShapeDtypeStruct((t_loc, d_model), jnp.float32),
        in_specs=[
            pl.BlockSpec(memory_space=pl.ANY),
            pl.BlockSpec(memory_space=pl.ANY),
            pl.BlockSpec(memory_space=pl.ANY),
            pl.BlockSpec(memory_space=pl.ANY),
        ],
        out_specs=pl.BlockSpec(memory_space=pltpu.VMEM),
        scratch_shapes=[
            pltpu.VMEM((n_tok, d_model), jnp.float32),
            pltpu.VMEM((N_Z, n_tok, e_loc), jnp.float32),
            pltpu.VMEM((2, d_model, f_dim), jnp.float32),
            pltpu.VMEM((2, f_dim, d_model), jnp.float32),
            pltpu.VMEM((n_tok, d_model), jnp.float32),
            pltpu.VMEM((t_loc, d_model), jnp.float32),
            pltpu.VMEM((N_Z - 1, t_loc, d_model), jnp.float32),
            pltpu.SemaphoreType.DMA((N_Z - 1,)),
            pltpu.SemaphoreType.DMA((N_Z - 1,)),
            pltpu.SemaphoreType.DMA((N_Z - 1,)),
            pltpu.SemaphoreType.DMA((N_Z - 1,)),
            pltpu.SemaphoreType.DMA((N_Z - 1,)),
            pltpu.SemaphoreType.DMA((N_Z - 1,)),
            pltpu.SemaphoreType.DMA,
            pltpu.SemaphoreType.DMA,
            pltpu.SemaphoreType.DMA((2,)),
            pltpu.SemaphoreType.DMA((2,)),
        ],
        compiler_params=pltpu.CompilerParams(
            collective_id=0, vmem_limit_bytes=63 * 1024 * 1024),
    )(x, router, W1, W2)
